# baseline (speedup 1.0000x reference)
import functools

import jax
import jax.numpy as jnp
from jax import lax
from jax.experimental import pallas as pl
from jax.experimental.pallas import tpu as pltpu
from jax.experimental.pallas import tpu_sc as plsc

N_NODES = 10000
T = 8
E = 640000
TT = T * T

NC = 2
NS = 16
NW = NC * NS
EPW = E // NW
C = 80
NCHUNK = EPW // C
G = C // 16

_mesh = plsc.VectorSubcoreMesh(
    core_axis_name="c", subcore_axis_name="s", num_cores=NC, num_subcores=NS
)


@functools.partial(
    pl.kernel,
    out_type=jax.ShapeDtypeStruct((E, TT), jnp.float32),
    mesh=_mesh,
    compiler_params=pltpu.CompilerParams(needs_layout_passes=False),
    scratch_types=[
        pltpu.VMEM((N_NODES * T,), jnp.float32),
        pltpu.VMEM((C,), jnp.int32),
        pltpu.VMEM((C,), jnp.int32),
        pltpu.VMEM((C, TT), jnp.float32),
    ],
)
def _encode(src_hbm, dst_hbm, node_hbm, out_hbm, table_v, idx1_v, idx2_v, out_v):
    wid = lax.axis_index("s") * NC + lax.axis_index("c")
    pltpu.sync_copy(node_hbm, table_v)

    def chunk_body(k, _):
        base = wid * EPW + k * C
        pltpu.sync_copy(src_hbm.at[pl.ds(base, C)], idx1_v)
        pltpu.sync_copy(dst_hbm.at[pl.ds(base, C)], idx2_v)

        @plsc.parallel_loop(0, G, 1, unroll=1)
        def group_body(g):
            lane = lax.iota(jnp.int32, 16)
            pat_b = lane % T
            pat_a = [2 * r + lane // T for r in range(4)]
            vs8 = idx1_v[pl.ds(g * 16, 16)] * T
            vd8 = idx2_v[pl.ds(g * 16, 16)] * T
            obase = g * 16
            for l in range(16):
                sb = jnp.full((16,), vs8[l], jnp.int32)
                db = jnp.full((16,), vd8[l], jnp.int32)
                b = plsc.load_gather(table_v, [db + pat_b])
                for r in range(4):
                    a = plsc.load_gather(table_v, [sb + pat_a[r]])
                    out_v[obase + l, pl.ds(r * 16, 16)] = a * b

        pltpu.sync_copy(out_v, out_hbm.at[pl.ds(base, C)])
        return 0

    lax.fori_loop(0, NCHUNK, chunk_body, 0)


def kernel(edge_index, node_type):
    src = edge_index[0]
    dst = edge_index[1]
    return _encode(src, dst, node_type.reshape(-1))


# speedup vs baseline: 1.3849x; 1.3849x over previous
import functools

import jax
import jax.numpy as jnp
from jax import lax
from jax.experimental import pallas as pl
from jax.experimental.pallas import tpu as pltpu
from jax.experimental.pallas import tpu_sc as plsc

N_NODES = 10000
T = 8
E = 640000
TT = T * T

NC = 2
NS = 16
NW = NC * NS
EPW = E // NW
C = 192
NFULL = EPW // C
NPAIR = NFULL // 2
CTAIL = EPW - NFULL * C

_mesh = plsc.VectorSubcoreMesh(
    core_axis_name="c", subcore_axis_name="s", num_cores=NC, num_subcores=NS
)


@functools.partial(
    pl.kernel,
    out_type=jax.ShapeDtypeStruct((E, TT), jnp.float32),
    mesh=_mesh,
    compiler_params=pltpu.CompilerParams(needs_layout_passes=False),
    scratch_types=[
        pltpu.VMEM((N_NODES * T,), jnp.float32),
        pltpu.VMEM((C,), jnp.int32),
        pltpu.VMEM((C,), jnp.int32),
        pltpu.VMEM((C,), jnp.int32),
        pltpu.VMEM((C,), jnp.int32),
        pltpu.VMEM((C, TT), jnp.float32),
        pltpu.VMEM((C, TT), jnp.float32),
        pltpu.SemaphoreType.DMA,
        pltpu.SemaphoreType.DMA,
    ],
)
def _encode(src_hbm, dst_hbm, node_hbm, out_hbm,
            table_v, idx1a, idx2a, idx1b, idx2b, outa, outb, sema, semb):
    wid = lax.axis_index("s") * NC + lax.axis_index("c")
    pltpu.sync_copy(node_hbm, table_v)
    idx_bufs = ((idx1a, idx2a), (idx1b, idx2b))
    out_bufs = (outa, outb)
    sems = (sema, semb)

    def compute_chunk(idx1_v, idx2_v, out_v, n_edges):
        @plsc.parallel_loop(0, n_edges // 16, 1, unroll=1)
        def group_body(g):
            lane = lax.iota(jnp.int32, 16)
            pat_b = lane % T
            pat_a = [2 * r + lane // T for r in range(4)]
            vs8 = idx1_v[pl.ds(g * 16, 16)] * T
            vd8 = idx2_v[pl.ds(g * 16, 16)] * T
            obase = g * 16
            for l in range(16):
                sb = jnp.full((16,), vs8[l], jnp.int32)
                db = jnp.full((16,), vd8[l], jnp.int32)
                b = plsc.load_gather(table_v, [db + pat_b])
                for r in range(4):
                    a = plsc.load_gather(table_v, [sb + pat_a[r]])
                    out_v[obase + l, pl.ds(r * 16, 16)] = a * b

    def pair_body(k2, _):
        for bi in range(2):
            k = k2 * 2 + bi
            base = wid * EPW + k * C
            idx1_v, idx2_v = idx_bufs[bi]
            out_v = out_bufs[bi]
            sem = sems[bi]
            pltpu.sync_copy(src_hbm.at[pl.ds(base, C)], idx1_v)
            pltpu.sync_copy(dst_hbm.at[pl.ds(base, C)], idx2_v)

            @pl.when(k2 > 0)
            def _wait_prev():
                pltpu.make_async_copy(
                    out_v, out_hbm.at[pl.ds(wid * EPW, C)], sem
                ).wait()

            compute_chunk(idx1_v, idx2_v, out_v, C)
            pltpu.async_copy(out_v, out_hbm.at[pl.ds(base, C)], sem)
        return 0

    lax.fori_loop(0, NPAIR, pair_body, 0)
    for bi in range(2):
        pltpu.make_async_copy(
            out_bufs[bi], out_hbm.at[pl.ds(wid * EPW, C)], sems[bi]
        ).wait()

    tbase = wid * EPW + NFULL * C
    idx1_v, idx2_v = idx_bufs[0]
    out_v = out_bufs[0]
    pltpu.sync_copy(src_hbm.at[pl.ds(tbase, CTAIL)], idx1_v.at[pl.ds(0, CTAIL)])
    pltpu.sync_copy(dst_hbm.at[pl.ds(tbase, CTAIL)], idx2_v.at[pl.ds(0, CTAIL)])
    compute_chunk(idx1_v, idx2_v, out_v, CTAIL)
    pltpu.sync_copy(out_v.at[pl.ds(0, CTAIL)], out_hbm.at[pl.ds(tbase, CTAIL)])


def kernel(edge_index, node_type):
    src = edge_index[0]
    dst = edge_index[1]
    return _encode(src, dst, node_type.reshape(-1))


# speedup vs baseline: 6.8524x; 4.9480x over previous
import functools

import jax
import jax.numpy as jnp
from jax import lax
from jax.experimental import pallas as pl
from jax.experimental.pallas import tpu as pltpu
from jax.experimental.pallas import tpu_sc as plsc

N_NODES = 10000
T = 8
E = 640000
TT = T * T

NC = 2
NS = 16
NW = NC * NS
C = 128
NCHUNKS = E // C
KPW = NCHUNKS // NW
NREM = NCHUNKS - KPW * NW
NBUF = 4
NQUAD = KPW // NBUF

_mesh = plsc.VectorSubcoreMesh(
    core_axis_name="c", subcore_axis_name="s", num_cores=NC, num_subcores=NS
)


@functools.partial(
    pl.kernel,
    out_type=jax.ShapeDtypeStruct((TT, E), jnp.float32),
    mesh=_mesh,
    compiler_params=pltpu.CompilerParams(needs_layout_passes=False),
    scratch_types=[
        pltpu.VMEM((N_NODES * T,), jnp.float32),
        [pltpu.VMEM((C,), jnp.int32) for _ in range(NBUF)],
        [pltpu.VMEM((C,), jnp.int32) for _ in range(NBUF)],
        [pltpu.VMEM((TT, C), jnp.float32) for _ in range(NBUF)],
        [pltpu.SemaphoreType.DMA for _ in range(NBUF)],
        [pltpu.SemaphoreType.DMA for _ in range(NBUF)],
    ],
)
def _encode(src_hbm, dst_hbm, node_hbm, out_hbm,
            table_v, idx1s, idx2s, outs, isems, osems):
    wid = lax.axis_index("s") * NC + lax.axis_index("c")
    pltpu.sync_copy(node_hbm, table_v)

    def chunk_base(k):
        return (wid + k * NW) * C

    def start_idx(k, bi):
        base = chunk_base(k)
        pltpu.async_copy(src_hbm.at[pl.ds(base, C)], idx1s[bi], isems[bi])
        pltpu.async_copy(dst_hbm.at[pl.ds(base, C)], idx2s[bi], isems[bi])

    def wait_idx(bi):
        pltpu.make_async_copy(src_hbm.at[pl.ds(0, C)], idx1s[bi], isems[bi]).wait()
        pltpu.make_async_copy(dst_hbm.at[pl.ds(0, C)], idx2s[bi], isems[bi]).wait()

    def wait_out(bi):
        pltpu.make_async_copy(
            outs[bi], out_hbm.at[:, pl.ds(0, C)], osems[bi]
        ).wait()

    def compute_chunk(idx1_v, idx2_v, out_v):
        @plsc.parallel_loop(0, C // 16, 1, unroll=1)
        def group_body(g):
            vs8 = idx1_v[pl.ds(g * 16, 16)] * T
            vd8 = idx2_v[pl.ds(g * 16, 16)] * T
            a = [plsc.load_gather(table_v, [vs8 + i]) for i in range(T)]
            b = [plsc.load_gather(table_v, [vd8 + j]) for j in range(T)]
            for i in range(T):
                for j in range(T):
                    out_v[i * T + j, pl.ds(g * 16, 16)] = a[i] * b[j]

    for bi in range(NBUF):
        start_idx(bi, bi)

    def quad_body(k4, _):
        for bi in range(NBUF):
            k = k4 * NBUF + bi
            wait_idx(bi)

            @pl.when(k4 > 0)
            def _wait_prev():
                wait_out(bi)

            compute_chunk(idx1s[bi], idx2s[bi], outs[bi])
            pltpu.async_copy(
                outs[bi], out_hbm.at[:, pl.ds(chunk_base(k), C)], osems[bi]
            )

            @pl.when(k4 < NQUAD - 1)
            def _prefetch():
                start_idx(k + NBUF, bi)
        return 0

    lax.fori_loop(0, NQUAD, quad_body, 0)
    for bi in range(NBUF):
        wait_out(bi)

    @pl.when(wid < NREM)
    def _leftover():
        base = (KPW * NW + wid) * C
        pltpu.sync_copy(src_hbm.at[pl.ds(base, C)], idx1s[0])
        pltpu.sync_copy(dst_hbm.at[pl.ds(base, C)], idx2s[0])
        compute_chunk(idx1s[0], idx2s[0], outs[0])
        pltpu.sync_copy(outs[0], out_hbm.at[:, pl.ds(base, C)])


def kernel(edge_index, node_type):
    src = edge_index[0]
    dst = edge_index[1]
    out_t = _encode(src, dst, node_type.reshape(-1))
    return out_t.T


# speedup vs baseline: 7.2515x; 1.0582x over previous
import functools

import jax
import jax.numpy as jnp
from jax import lax
from jax.experimental import pallas as pl
from jax.experimental.pallas import tpu as pltpu
from jax.experimental.pallas import tpu_sc as plsc

N_NODES = 10000
T = 8
E = 640000
TT = T * T

NC = 2
NS = 16
NW = NC * NS
C = 128
NCHUNKS = E // C
KPW = NCHUNKS // NW
NREM = NCHUNKS - KPW * NW
NBUF = 4
NQUAD = KPW // NBUF

_mesh = plsc.VectorSubcoreMesh(
    core_axis_name="c", subcore_axis_name="s", num_cores=NC, num_subcores=NS
)


@functools.partial(
    pl.kernel,
    out_type=jax.ShapeDtypeStruct((TT, E), jnp.float32),
    mesh=_mesh,
    compiler_params=pltpu.CompilerParams(needs_layout_passes=False),
    scratch_types=[
        pltpu.VMEM((N_NODES * T,), jnp.float32),
        [pltpu.VMEM((C,), jnp.int32) for _ in range(NBUF)],
        [pltpu.VMEM((C,), jnp.int32) for _ in range(NBUF)],
        [pltpu.VMEM((TT, C), jnp.float32) for _ in range(NBUF)],
        [pltpu.SemaphoreType.DMA for _ in range(NBUF)],
        [pltpu.SemaphoreType.DMA for _ in range(NBUF)],
    ],
)
def _encode(edge_hbm, node_hbm, out_hbm,
            table_v, idx1s, idx2s, outs, isems, osems):
    wid = lax.axis_index("s") * NC + lax.axis_index("c")
    pltpu.sync_copy(node_hbm, table_v)

    def chunk_base(k):
        return (wid + k * NW) * C

    def start_idx(k, bi):
        base = chunk_base(k)
        pltpu.async_copy(edge_hbm.at[pl.ds(base, C)], idx1s[bi], isems[bi])
        pltpu.async_copy(edge_hbm.at[pl.ds(E + base, C)], idx2s[bi], isems[bi])

    def wait_idx(bi):
        pltpu.make_async_copy(edge_hbm.at[pl.ds(0, C)], idx1s[bi], isems[bi]).wait()
        pltpu.make_async_copy(edge_hbm.at[pl.ds(0, C)], idx2s[bi], isems[bi]).wait()

    def wait_out(bi):
        pltpu.make_async_copy(
            outs[bi], out_hbm.at[:, pl.ds(0, C)], osems[bi]
        ).wait()

    def compute_chunk(idx1_v, idx2_v, out_v):
        @plsc.parallel_loop(0, C // 16, 1, unroll=1)
        def group_body(g):
            vs8 = idx1_v[pl.ds(g * 16, 16)] * T
            vd8 = idx2_v[pl.ds(g * 16, 16)] * T
            a = [plsc.load_gather(table_v, [vs8 + i]) for i in range(T)]
            b = [plsc.load_gather(table_v, [vd8 + j]) for j in range(T)]
            for i in range(T):
                for j in range(T):
                    out_v[i * T + j, pl.ds(g * 16, 16)] = a[i] * b[j]

    for bi in range(NBUF):
        start_idx(bi, bi)

    def quad_body(k4, _):
        for bi in range(NBUF):
            k = k4 * NBUF + bi
            wait_idx(bi)

            @pl.when(k4 > 0)
            def _wait_prev():
                wait_out(bi)

            compute_chunk(idx1s[bi], idx2s[bi], outs[bi])
            pltpu.async_copy(
                outs[bi], out_hbm.at[:, pl.ds(chunk_base(k), C)], osems[bi]
            )

            @pl.when(k4 < NQUAD - 1)
            def _prefetch():
                start_idx(k + NBUF, bi)
        return 0

    lax.fori_loop(0, NQUAD, quad_body, 0)
    for bi in range(NBUF):
        wait_out(bi)

    @pl.when(wid < NREM)
    def _leftover():
        base = (KPW * NW + wid) * C
        pltpu.sync_copy(edge_hbm.at[pl.ds(base, C)], idx1s[0])
        pltpu.sync_copy(edge_hbm.at[pl.ds(E + base, C)], idx2s[0])
        compute_chunk(idx1s[0], idx2s[0], outs[0])
        pltpu.sync_copy(outs[0], out_hbm.at[:, pl.ds(base, C)])


def kernel(edge_index, node_type):
    out_t = _encode(edge_index.reshape(-1), node_type.reshape(-1))
    return out_t.T
